# baseline (speedup 1.0000x reference)
import functools

import jax
import jax.numpy as jnp
from jax import lax
from jax.experimental import pallas as pl
from jax.experimental.pallas import tpu as pltpu
from jax.experimental.pallas import tpu_sc as plsc

_TEMP = 0.05
_C = 1000
_CP = 1024
_SL = 64
_N = 100000
_F = 128
_CH = 80
_NB = 4000
_NT = 48000
_NCHUNKS = (_N - _NT) // _CH
_SC_T0 = _NT // _CH
_NW = 32
_NG = 16
_BASE = _NCHUNKS // _NW
_REM = _NCHUNKS % _NW
_B = 1024


def _sc_body(feat_hbm, lab_hbm, cs_out, cnt_out,
             in0, in1, lb0, lb1, st0, st1, lbs0, lbs1, cnt,
             acc_sh,
             si0, si1, sl0, sl1, ss0, ss1):
    c = lax.axis_index("c")
    s = lax.axis_index("s")
    wid = s * 2 + c
    start = _SC_T0 + wid * _BASE + jnp.minimum(wid, _REM)
    nch = _BASE + jnp.where(wid < _REM, 1, 0)

    iota = lax.iota(jnp.int32, 16)
    zero16 = jnp.zeros((16,), jnp.float32)
    ones16 = jnp.ones((16,), jnp.float32)

    lane0 = iota == 0

    def zrow(i, _):
        for k in range(4):
            for j in range(_F // 16):
                st0[i * 4 + k, pl.ds(16 * j, 16)] = zero16
        return 0

    lax.fori_loop(0, _CH // 4, zrow, 0)
    pltpu.sync_copy(st0.at[pl.ds(0, _SL)], acc_sh.at[pl.ds(s * _SL, _SL)])
    for i in range(_CP // 16):
        cnt[pl.ds(16 * i, 16)] = zero16
    plsc.subcore_barrier()

    def _start(t, fbuf, lbuf, fsem, lsem):
        r0 = t * _CH
        pltpu.async_copy(feat_hbm.at[pl.ds(r0, _CH)], fbuf, fsem)
        pltpu.async_copy(lab_hbm.at[pl.ds(r0, _CH)], lbuf, lsem)

    def _wait_in(fbuf, lbuf, fsem, lsem):
        pltpu.make_async_copy(feat_hbm.at[pl.ds(0, _CH)], fbuf, fsem).wait()
        pltpu.make_async_copy(lab_hbm.at[pl.ds(0, _CH)], lbuf, lsem).wait()

    def _wait_scat(sbuf, lbsbuf, ssem):
        pltpu.make_async_copy(sbuf, acc_sh.at[lbsbuf], ssem).wait()

    def _fire_scat(sbuf, lbsbuf, ssem):
        pltpu.async_copy(sbuf, acc_sh.at[lbsbuf], ssem, add=True)

    def _process(fbuf, lbuf, sbuf, lbsbuf):
        for j in range(_CH // 16):
            lbsbuf[pl.ds(16 * j, 16)] = lbuf[pl.ds(16 * j, 16)]

        def sub(kb, _):
            labs16 = lbuf[pl.ds(kb * 16, 16)]
            for rr in range(16):
                r = kb * 16 + rr
                labsp = labs16.at[jnp.full((16,), rr, jnp.int32)].get(
                    mode="promise_in_bounds")
                plsc.addupdate_scatter(cnt, [labsp], ones16, mask=lane0)
                v = [fbuf[r, pl.ds(16 * j, 16)] for j in range(8)]
                sq = [x * x for x in v]
                tot = (((sq[0] + sq[1]) + (sq[2] + sq[3]))
                       + ((sq[4] + sq[5]) + (sq[6] + sq[7])))
                for k in (1, 2, 4, 8):
                    tot = tot + tot.at[iota ^ k].get(mode="promise_in_bounds")
                tot = jnp.maximum(tot, 1e-24)
                ib = lax.bitcast_convert_type(tot, jnp.int32)
                yi = jnp.int32(0x5F3759DF) - lax.shift_right_logical(ib, 1)
                yv = lax.bitcast_convert_type(yi, jnp.float32)
                h = 0.5 * tot
                for _ in range(2):
                    yv = yv * (1.5 - h * yv * yv)
                for j in range(8):
                    sbuf[r, pl.ds(16 * j, 16)] = v[j] * yv
            return 0

        lax.fori_loop(0, _CH // 16, sub, 0)

    _start(start, in0, lb0, si0, sl0)
    _start(start + 1, in1, lb1, si1, sl1)

    def pair(p, _):
        t0 = start + 2 * p
        _wait_in(in0, lb0, si0, sl0)

        @pl.when(p > 0)
        def _():
            _wait_scat(st0, lbs0, ss0)

        _process(in0, lb0, st0, lbs0)

        @pl.when(2 * p + 2 < nch)
        def _():
            _start(t0 + 2, in0, lb0, si0, sl0)

        _fire_scat(st0, lbs0, ss0)

        @pl.when(2 * p + 1 < nch)
        def _():
            _wait_in(in1, lb1, si1, sl1)

            @pl.when(p > 0)
            def _():
                _wait_scat(st1, lbs1, ss1)

            _process(in1, lb1, st1, lbs1)

            @pl.when(2 * p + 3 < nch)
            def _():
                _start(t0 + 3, in1, lb1, si1, sl1)

            _fire_scat(st1, lbs1, ss1)

        return 0

    lax.fori_loop(0, (nch + 1) // 2, pair, 0)

    _wait_scat(st0, lbs0, ss0)
    _wait_scat(st1, lbs1, ss1)
    plsc.subcore_barrier()

    pltpu.sync_copy(acc_sh.at[pl.ds(s * _SL, _SL)],
                    cs_out.at[c, pl.ds(s * _SL, _SL)])
    pltpu.sync_copy(cnt, cnt_out.at[c, s])


def _segment_sums_sc(features, labels):
    mesh = plsc.VectorSubcoreMesh(core_axis_name="c", subcore_axis_name="s")
    k = pl.kernel(
        _sc_body,
        out_type=[
            jax.ShapeDtypeStruct((2, _CP, _F), jnp.float32),
            jax.ShapeDtypeStruct((2, _NG, _CP), jnp.float32),
        ],
        mesh=mesh,
        scratch_types=[
            pltpu.VMEM((_CH, _F), jnp.float32),
            pltpu.VMEM((_CH, _F), jnp.float32),
            pltpu.VMEM((_CH,), jnp.int32),
            pltpu.VMEM((_CH,), jnp.int32),
            pltpu.VMEM((_CH, _F), jnp.float32),
            pltpu.VMEM((_CH, _F), jnp.float32),
            pltpu.VMEM((_CH,), jnp.int32),
            pltpu.VMEM((_CH,), jnp.int32),
            pltpu.VMEM((_CP,), jnp.float32),
            pltpu.VMEM_SHARED((_CP, _F), jnp.float32),
            pltpu.SemaphoreType.DMA,
            pltpu.SemaphoreType.DMA,
            pltpu.SemaphoreType.DMA,
            pltpu.SemaphoreType.DMA,
            pltpu.SemaphoreType.DMA,
            pltpu.SemaphoreType.DMA,
        ],
        compiler_params=pltpu.CompilerParams(needs_layout_passes=False),
    )
    return k(features, labels)


def _seg_tc_body(lab_ref, feat_ref, cs_ref, cnt_ref):
    g = pl.program_id(0)
    fb = feat_ref[...]
    ss = jnp.sum(fb * fb, axis=1, keepdims=True)
    rn = lax.rsqrt(jnp.maximum(ss, 1e-24))
    fn = (fb * rn).astype(jnp.bfloat16)
    lab = lab_ref[0]
    cio = lax.broadcasted_iota(jnp.int32, (_CP, _NB), 0)
    oh = (cio == lab).astype(jnp.bfloat16)
    csb = lax.dot_general(oh, fn, (((1,), (0,)), ((), ())),
                          preferred_element_type=jnp.float32)
    ones8 = jnp.ones((8, _NB), jnp.bfloat16)
    cntb = lax.dot_general(ones8, oh, (((1,), (1,)), ((), ())),
                           preferred_element_type=jnp.float32)

    @pl.when(g == 0)
    def _init():
        cs_ref[...] = jnp.zeros_like(cs_ref)
        cnt_ref[...] = jnp.zeros_like(cnt_ref)

    cs_ref[...] += csb
    cnt_ref[...] += cntb


def _segment_sums_tc(features, labels):
    k = _NT // _NB
    labels_b = labels[:_NT].reshape(k, 1, _NB)
    return pl.pallas_call(
        _seg_tc_body,
        grid=(k,),
        in_specs=[
            pl.BlockSpec((1, 1, _NB), lambda g: (g, 0, 0)),
            pl.BlockSpec((_NB, _F), lambda g: (g, 0)),
        ],
        out_specs=[
            pl.BlockSpec((_CP, _F), lambda g: (0, 0)),
            pl.BlockSpec((8, _CP), lambda g: (0, 0)),
        ],
        out_shape=[
            jax.ShapeDtypeStruct((_CP, _F), jnp.float32),
            jax.ShapeDtypeStruct((8, _CP), jnp.float32),
        ],
    )(labels_b, features)


def _epi_body(in_ref, idx_ref, lab2_ref, cs_ref, cnt_ref,
              cst_ref, cntt_ref, out_ref):
    b = in_ref.shape[0]
    cs = (cs_ref[0] + cs_ref[1] + cst_ref[...])[:_C]
    cntrow = (jnp.sum(cnt_ref[...], axis=0)
              + cntt_ref[0:1, :])[:, :_C]

    u = in_ref[...]
    ss = jnp.sum(u * u, axis=1, keepdims=True)
    un = u * lax.rsqrt(jnp.maximum(ss, 1e-24))
    logits = lax.dot_general(un, cs, (((1,), (1,)), ((), ())),
                             preferred_element_type=jnp.float32)
    mask = cntrow > 0.0
    denom = jnp.where(mask, cntrow, 1.0)
    sim = logits / (_TEMP * denom)
    exps = jnp.exp(sim) * mask.astype(jnp.float32)
    sums = jnp.sum(exps, axis=1, keepdims=True) + 1e-6
    logp = jnp.log(exps / sums + 1e-6)
    idx = idx_ref[...]
    q = idx // _C
    r = idx - q * _C
    l2 = lab2_ref[...].astype(jnp.float32)
    qio = lax.broadcasted_iota(jnp.int32, (b, l2.shape[0]), 1)
    ohq = (qio == q).astype(jnp.float32)
    rowsel = lax.dot_general(ohq, l2, (((1,), (0,)), ((), ())),
                             preferred_element_type=jnp.float32)
    rio = lax.broadcasted_iota(jnp.int32, (b, _C), 1)
    ohr = (rio == r).astype(jnp.float32)
    tcol = jnp.sum(rowsel * ohr, axis=1, keepdims=True)
    cio = lax.broadcasted_iota(jnp.int32, (b, _C), 1)
    oht = (cio == tcol.astype(jnp.int32)).astype(jnp.float32)
    loss = -jnp.sum(logp * oht) / float(b)
    out_ref[...] = jnp.full((1, 1), loss, jnp.float32)


def kernel(inputs, indexes, features, labels):
    n, f = features.shape
    b = inputs.shape[0]

    cs_parts, cnt_parts = _segment_sums_sc(features, labels)
    cs_tc, cnt_tc = _segment_sums_tc(features, labels)
    cnt3 = cnt_parts.reshape(32, 1, _CP)

    args = (inputs, indexes.reshape(b, 1), labels.reshape(n // _C, _C),
            cs_parts, cnt3, cs_tc, cnt_tc)
    loss = pl.pallas_call(
        _epi_body,
        in_specs=[pl.BlockSpec(a.shape, functools.partial(
                      lambda r: (0,) * r, len(a.shape))) for a in args],
        out_specs=pl.BlockSpec((1, 1), lambda: (0, 0)),
        out_shape=jax.ShapeDtypeStruct((1, 1), jnp.float32),
    )(*args)

    return loss[0, 0]


# speedup vs baseline: 1.0642x; 1.0642x over previous
import functools

import jax
import jax.numpy as jnp
from jax import lax
from jax.experimental import pallas as pl
from jax.experimental.pallas import tpu as pltpu
from jax.experimental.pallas import tpu_sc as plsc

_TEMP = 0.05
_C = 1000
_CP = 1024
_SL = 64
_N = 100000
_F = 128
_CH = 80
_NB = 4000
_NT = 52000
_NCHUNKS = (_N - _NT) // _CH
_SC_T0 = _NT // _CH
_NW = 32
_NG = 16
_BASE = _NCHUNKS // _NW
_REM = _NCHUNKS % _NW
_B = 1024


def _sc_body(feat_hbm, lab_hbm, cs_out, cnt_out,
             in0, in1, lb0, lb1, st0, st1, lbs0, lbs1, cnt,
             acc_sh,
             si0, si1, sl0, sl1, ss0, ss1):
    c = lax.axis_index("c")
    s = lax.axis_index("s")
    wid = s * 2 + c
    start = _SC_T0 + wid * _BASE + jnp.minimum(wid, _REM)
    nch = _BASE + jnp.where(wid < _REM, 1, 0)

    iota = lax.iota(jnp.int32, 16)
    zero16 = jnp.zeros((16,), jnp.float32)
    ones16 = jnp.ones((16,), jnp.float32)

    lane0 = iota == 0

    def zrow(i, _):
        for k in range(4):
            for j in range(_F // 16):
                st0[i * 4 + k, pl.ds(16 * j, 16)] = zero16
        return 0

    lax.fori_loop(0, _CH // 4, zrow, 0)
    pltpu.sync_copy(st0.at[pl.ds(0, _SL)], acc_sh.at[pl.ds(s * _SL, _SL)])
    for i in range(_CP // 16):
        cnt[pl.ds(16 * i, 16)] = zero16
    plsc.subcore_barrier()

    def _start(t, fbuf, lbuf, fsem, lsem):
        r0 = t * _CH
        pltpu.async_copy(feat_hbm.at[pl.ds(r0, _CH)], fbuf, fsem)
        pltpu.async_copy(lab_hbm.at[pl.ds(r0, _CH)], lbuf, lsem)

    def _wait_in(fbuf, lbuf, fsem, lsem):
        pltpu.make_async_copy(feat_hbm.at[pl.ds(0, _CH)], fbuf, fsem).wait()
        pltpu.make_async_copy(lab_hbm.at[pl.ds(0, _CH)], lbuf, lsem).wait()

    def _wait_scat(sbuf, lbsbuf, ssem):
        pltpu.make_async_copy(sbuf, acc_sh.at[lbsbuf], ssem).wait()

    def _fire_scat(sbuf, lbsbuf, ssem):
        pltpu.async_copy(sbuf, acc_sh.at[lbsbuf], ssem, add=True)

    def _process(fbuf, lbuf, sbuf, lbsbuf):
        for j in range(_CH // 16):
            lbsbuf[pl.ds(16 * j, 16)] = lbuf[pl.ds(16 * j, 16)]

        def sub(kb, _):
            labs16 = lbuf[pl.ds(kb * 16, 16)]
            for rr in range(16):
                r = kb * 16 + rr
                labsp = labs16.at[jnp.full((16,), rr, jnp.int32)].get(
                    mode="promise_in_bounds")
                plsc.addupdate_scatter(cnt, [labsp], ones16, mask=lane0)
                v = [fbuf[r, pl.ds(16 * j, 16)] for j in range(8)]
                sq = [x * x for x in v]
                tot = (((sq[0] + sq[1]) + (sq[2] + sq[3]))
                       + ((sq[4] + sq[5]) + (sq[6] + sq[7])))
                for k in (1, 2, 4, 8):
                    tot = tot + tot.at[iota ^ k].get(mode="promise_in_bounds")
                tot = jnp.maximum(tot, 1e-24)
                ib = lax.bitcast_convert_type(tot, jnp.int32)
                yi = jnp.int32(0x5F3759DF) - lax.shift_right_logical(ib, 1)
                yv = lax.bitcast_convert_type(yi, jnp.float32)
                h = 0.5 * tot
                for _ in range(2):
                    yv = yv * (1.5 - h * yv * yv)
                for j in range(8):
                    sbuf[r, pl.ds(16 * j, 16)] = v[j] * yv
            return 0

        lax.fori_loop(0, _CH // 16, sub, 0)

    _start(start, in0, lb0, si0, sl0)
    _start(start + 1, in1, lb1, si1, sl1)

    def pair(p, _):
        t0 = start + 2 * p
        _wait_in(in0, lb0, si0, sl0)

        @pl.when(p > 0)
        def _():
            _wait_scat(st0, lbs0, ss0)

        _process(in0, lb0, st0, lbs0)

        @pl.when(2 * p + 2 < nch)
        def _():
            _start(t0 + 2, in0, lb0, si0, sl0)

        _fire_scat(st0, lbs0, ss0)

        @pl.when(2 * p + 1 < nch)
        def _():
            _wait_in(in1, lb1, si1, sl1)

            @pl.when(p > 0)
            def _():
                _wait_scat(st1, lbs1, ss1)

            _process(in1, lb1, st1, lbs1)

            @pl.when(2 * p + 3 < nch)
            def _():
                _start(t0 + 3, in1, lb1, si1, sl1)

            _fire_scat(st1, lbs1, ss1)

        return 0

    lax.fori_loop(0, (nch + 1) // 2, pair, 0)

    _wait_scat(st0, lbs0, ss0)
    _wait_scat(st1, lbs1, ss1)
    plsc.subcore_barrier()

    pltpu.sync_copy(acc_sh.at[pl.ds(s * _SL, _SL)],
                    cs_out.at[c, pl.ds(s * _SL, _SL)])
    pltpu.sync_copy(cnt, cnt_out.at[c, s])


def _segment_sums_sc(features, labels):
    mesh = plsc.VectorSubcoreMesh(core_axis_name="c", subcore_axis_name="s")
    k = pl.kernel(
        _sc_body,
        out_type=[
            jax.ShapeDtypeStruct((2, _CP, _F), jnp.float32),
            jax.ShapeDtypeStruct((2, _NG, _CP), jnp.float32),
        ],
        mesh=mesh,
        scratch_types=[
            pltpu.VMEM((_CH, _F), jnp.float32),
            pltpu.VMEM((_CH, _F), jnp.float32),
            pltpu.VMEM((_CH,), jnp.int32),
            pltpu.VMEM((_CH,), jnp.int32),
            pltpu.VMEM((_CH, _F), jnp.float32),
            pltpu.VMEM((_CH, _F), jnp.float32),
            pltpu.VMEM((_CH,), jnp.int32),
            pltpu.VMEM((_CH,), jnp.int32),
            pltpu.VMEM((_CP,), jnp.float32),
            pltpu.VMEM_SHARED((_CP, _F), jnp.float32),
            pltpu.SemaphoreType.DMA,
            pltpu.SemaphoreType.DMA,
            pltpu.SemaphoreType.DMA,
            pltpu.SemaphoreType.DMA,
            pltpu.SemaphoreType.DMA,
            pltpu.SemaphoreType.DMA,
        ],
        compiler_params=pltpu.CompilerParams(needs_layout_passes=False),
    )
    return k(features, labels)


def _seg_tc_body(lab_ref, feat_ref, cs_ref, cnt_ref):
    g = pl.program_id(0)
    fb = feat_ref[...]
    ss = jnp.sum(fb * fb, axis=1, keepdims=True)
    rn = lax.rsqrt(jnp.maximum(ss, 1e-24))
    fn = (fb * rn).astype(jnp.bfloat16)
    lab = lab_ref[0]
    cio = lax.broadcasted_iota(jnp.int32, (_CP, _NB), 0)
    oh = (cio == lab).astype(jnp.bfloat16)
    csb = lax.dot_general(oh, fn, (((1,), (0,)), ((), ())),
                          preferred_element_type=jnp.float32)
    ones8 = jnp.ones((8, _NB), jnp.bfloat16)
    cntb = lax.dot_general(ones8, oh, (((1,), (1,)), ((), ())),
                           preferred_element_type=jnp.float32)

    @pl.when(g == 0)
    def _init():
        cs_ref[...] = jnp.zeros_like(cs_ref)
        cnt_ref[...] = jnp.zeros_like(cnt_ref)

    cs_ref[...] += csb
    cnt_ref[...] += cntb


def _segment_sums_tc(features, labels):
    k = _NT // _NB
    labels_b = labels[:_NT].reshape(k, 1, _NB)
    return pl.pallas_call(
        _seg_tc_body,
        grid=(k,),
        in_specs=[
            pl.BlockSpec((1, 1, _NB), lambda g: (g, 0, 0)),
            pl.BlockSpec((_NB, _F), lambda g: (g, 0)),
        ],
        out_specs=[
            pl.BlockSpec((_CP, _F), lambda g: (0, 0)),
            pl.BlockSpec((8, _CP), lambda g: (0, 0)),
        ],
        out_shape=[
            jax.ShapeDtypeStruct((_CP, _F), jnp.float32),
            jax.ShapeDtypeStruct((8, _CP), jnp.float32),
        ],
    )(labels_b, features)


def _epi_body(in_ref, idx_ref, lab2_ref, cs_ref, cnt_ref,
              cst_ref, cntt_ref, out_ref):
    b = in_ref.shape[0]
    cs = (cs_ref[0] + cs_ref[1] + cst_ref[...])[:_C]
    cntrow = (jnp.sum(cnt_ref[...], axis=0)
              + cntt_ref[0:1, :])[:, :_C]

    u = in_ref[...]
    ss = jnp.sum(u * u, axis=1, keepdims=True)
    un = u * lax.rsqrt(jnp.maximum(ss, 1e-24))
    logits = lax.dot_general(un, cs, (((1,), (1,)), ((), ())),
                             preferred_element_type=jnp.float32)
    mask = cntrow > 0.0
    denom = jnp.where(mask, cntrow, 1.0)
    sim = logits / (_TEMP * denom)
    exps = jnp.exp(sim) * mask.astype(jnp.float32)
    sums = jnp.sum(exps, axis=1, keepdims=True) + 1e-6
    logp = jnp.log(exps / sums + 1e-6)
    idx = idx_ref[...]
    q = idx // _C
    r = idx - q * _C
    l2 = lab2_ref[...].astype(jnp.float32)
    qio = lax.broadcasted_iota(jnp.int32, (b, l2.shape[0]), 1)
    ohq = (qio == q).astype(jnp.float32)
    rowsel = lax.dot_general(ohq, l2, (((1,), (0,)), ((), ())),
                             preferred_element_type=jnp.float32)
    rio = lax.broadcasted_iota(jnp.int32, (b, _C), 1)
    ohr = (rio == r).astype(jnp.float32)
    tcol = jnp.sum(rowsel * ohr, axis=1, keepdims=True)
    cio = lax.broadcasted_iota(jnp.int32, (b, _C), 1)
    oht = (cio == tcol.astype(jnp.int32)).astype(jnp.float32)
    loss = -jnp.sum(logp * oht) / float(b)
    out_ref[...] = jnp.full((1, 1), loss, jnp.float32)


def kernel(inputs, indexes, features, labels):
    n, f = features.shape
    b = inputs.shape[0]

    cs_parts, cnt_parts = _segment_sums_sc(features, labels)
    cs_tc, cnt_tc = _segment_sums_tc(features, labels)
    cnt3 = cnt_parts.reshape(32, 1, _CP)

    args = (inputs, indexes.reshape(b, 1), labels.reshape(n // _C, _C),
            cs_parts, cnt3, cs_tc, cnt_tc)
    loss = pl.pallas_call(
        _epi_body,
        in_specs=[pl.BlockSpec(a.shape, functools.partial(
                      lambda r: (0,) * r, len(a.shape))) for a in args],
        out_specs=pl.BlockSpec((1, 1), lambda: (0, 0)),
        out_shape=jax.ShapeDtypeStruct((1, 1), jnp.float32),
    )(*args)

    return loss[0, 0]


# speedup vs baseline: 1.0787x; 1.0136x over previous
import functools

import jax
import jax.numpy as jnp
from jax import lax
from jax.experimental import pallas as pl
from jax.experimental.pallas import tpu as pltpu
from jax.experimental.pallas import tpu_sc as plsc

_TEMP = 0.05
_C = 1000
_CP = 1024
_SL = 64
_N = 100000
_F = 128
_CH = 160
_NB = 4000
_NT = 56000
_NCHUNKS = (_N - _NT) // _CH
_SC_T0 = _NT // _CH
_NW = 32
_NG = 16
_BASE = _NCHUNKS // _NW
_REM = _NCHUNKS % _NW
_B = 1024


def _sc_body(feat_hbm, lab_hbm, cs_out, cnt_out,
             in0, in1, lb0, lb1, st0, st1, lbs0, lbs1, cnt,
             acc_sh,
             si0, si1, sl0, sl1, ss0, ss1):
    c = lax.axis_index("c")
    s = lax.axis_index("s")
    wid = s * 2 + c
    start = _SC_T0 + wid * _BASE + jnp.minimum(wid, _REM)
    nch = _BASE + jnp.where(wid < _REM, 1, 0)

    iota = lax.iota(jnp.int32, 16)
    zero16 = jnp.zeros((16,), jnp.float32)
    ones16 = jnp.ones((16,), jnp.float32)

    lane0 = iota == 0

    def zrow(i, _):
        for k in range(4):
            for j in range(_F // 16):
                st0[i * 4 + k, pl.ds(16 * j, 16)] = zero16
        return 0

    lax.fori_loop(0, _CH // 4, zrow, 0)
    pltpu.sync_copy(st0.at[pl.ds(0, _SL)], acc_sh.at[pl.ds(s * _SL, _SL)])
    for i in range(_CP // 16):
        cnt[pl.ds(16 * i, 16)] = zero16
    plsc.subcore_barrier()

    def _start(t, fbuf, lbuf, fsem, lsem):
        r0 = t * _CH
        pltpu.async_copy(feat_hbm.at[pl.ds(r0, _CH)], fbuf, fsem)
        pltpu.async_copy(lab_hbm.at[pl.ds(r0, _CH)], lbuf, lsem)

    def _wait_in(fbuf, lbuf, fsem, lsem):
        pltpu.make_async_copy(feat_hbm.at[pl.ds(0, _CH)], fbuf, fsem).wait()
        pltpu.make_async_copy(lab_hbm.at[pl.ds(0, _CH)], lbuf, lsem).wait()

    def _wait_scat(sbuf, lbsbuf, ssem):
        pltpu.make_async_copy(sbuf, acc_sh.at[lbsbuf], ssem).wait()

    def _fire_scat(sbuf, lbsbuf, ssem):
        pltpu.async_copy(sbuf, acc_sh.at[lbsbuf], ssem, add=True)

    def _process(fbuf, lbuf, sbuf, lbsbuf):
        for j in range(_CH // 16):
            lbsbuf[pl.ds(16 * j, 16)] = lbuf[pl.ds(16 * j, 16)]

        def sub(kb, _):
            labs16 = lbuf[pl.ds(kb * 16, 16)]
            for rr in range(16):
                r = kb * 16 + rr
                labsp = labs16.at[jnp.full((16,), rr, jnp.int32)].get(
                    mode="promise_in_bounds")
                plsc.addupdate_scatter(cnt, [labsp], ones16, mask=lane0)
                v = [fbuf[r, pl.ds(16 * j, 16)] for j in range(8)]
                sq = [x * x for x in v]
                tot = (((sq[0] + sq[1]) + (sq[2] + sq[3]))
                       + ((sq[4] + sq[5]) + (sq[6] + sq[7])))
                for k in (1, 2, 4, 8):
                    tot = tot + tot.at[iota ^ k].get(mode="promise_in_bounds")
                tot = jnp.maximum(tot, 1e-24)
                ib = lax.bitcast_convert_type(tot, jnp.int32)
                yi = jnp.int32(0x5F3759DF) - lax.shift_right_logical(ib, 1)
                yv = lax.bitcast_convert_type(yi, jnp.float32)
                h = 0.5 * tot
                for _ in range(2):
                    yv = yv * (1.5 - h * yv * yv)
                for j in range(8):
                    sbuf[r, pl.ds(16 * j, 16)] = v[j] * yv
            return 0

        lax.fori_loop(0, _CH // 16, sub, 0)

    _start(start, in0, lb0, si0, sl0)
    _start(start + 1, in1, lb1, si1, sl1)

    def pair(p, _):
        t0 = start + 2 * p
        _wait_in(in0, lb0, si0, sl0)

        @pl.when(p > 0)
        def _():
            _wait_scat(st0, lbs0, ss0)

        _process(in0, lb0, st0, lbs0)

        @pl.when(2 * p + 2 < nch)
        def _():
            _start(t0 + 2, in0, lb0, si0, sl0)

        _fire_scat(st0, lbs0, ss0)

        @pl.when(2 * p + 1 < nch)
        def _():
            _wait_in(in1, lb1, si1, sl1)

            @pl.when(p > 0)
            def _():
                _wait_scat(st1, lbs1, ss1)

            _process(in1, lb1, st1, lbs1)

            @pl.when(2 * p + 3 < nch)
            def _():
                _start(t0 + 3, in1, lb1, si1, sl1)

            _fire_scat(st1, lbs1, ss1)

        return 0

    lax.fori_loop(0, (nch + 1) // 2, pair, 0)

    _wait_scat(st0, lbs0, ss0)
    _wait_scat(st1, lbs1, ss1)
    plsc.subcore_barrier()

    pltpu.sync_copy(acc_sh.at[pl.ds(s * _SL, _SL)],
                    cs_out.at[c, pl.ds(s * _SL, _SL)])
    pltpu.sync_copy(cnt, cnt_out.at[c, s])


def _segment_sums_sc(features, labels):
    mesh = plsc.VectorSubcoreMesh(core_axis_name="c", subcore_axis_name="s")
    k = pl.kernel(
        _sc_body,
        out_type=[
            jax.ShapeDtypeStruct((2, _CP, _F), jnp.float32),
            jax.ShapeDtypeStruct((2, _NG, _CP), jnp.float32),
        ],
        mesh=mesh,
        scratch_types=[
            pltpu.VMEM((_CH, _F), jnp.float32),
            pltpu.VMEM((_CH, _F), jnp.float32),
            pltpu.VMEM((_CH,), jnp.int32),
            pltpu.VMEM((_CH,), jnp.int32),
            pltpu.VMEM((_CH, _F), jnp.float32),
            pltpu.VMEM((_CH, _F), jnp.float32),
            pltpu.VMEM((_CH,), jnp.int32),
            pltpu.VMEM((_CH,), jnp.int32),
            pltpu.VMEM((_CP,), jnp.float32),
            pltpu.VMEM_SHARED((_CP, _F), jnp.float32),
            pltpu.SemaphoreType.DMA,
            pltpu.SemaphoreType.DMA,
            pltpu.SemaphoreType.DMA,
            pltpu.SemaphoreType.DMA,
            pltpu.SemaphoreType.DMA,
            pltpu.SemaphoreType.DMA,
        ],
        compiler_params=pltpu.CompilerParams(needs_layout_passes=False),
    )
    return k(features, labels)


def _seg_tc_body(lab_ref, feat_ref, cs_ref, cnt_ref):
    g = pl.program_id(0)
    fb = feat_ref[...]
    ss = jnp.sum(fb * fb, axis=1, keepdims=True)
    rn = lax.rsqrt(jnp.maximum(ss, 1e-24))
    fn = (fb * rn).astype(jnp.bfloat16)
    lab = lab_ref[0]
    cio = lax.broadcasted_iota(jnp.int32, (_CP, _NB), 0)
    oh = (cio == lab).astype(jnp.bfloat16)
    csb = lax.dot_general(oh, fn, (((1,), (0,)), ((), ())),
                          preferred_element_type=jnp.float32)
    ones8 = jnp.ones((8, _NB), jnp.bfloat16)
    cntb = lax.dot_general(ones8, oh, (((1,), (1,)), ((), ())),
                           preferred_element_type=jnp.float32)

    @pl.when(g == 0)
    def _init():
        cs_ref[...] = jnp.zeros_like(cs_ref)
        cnt_ref[...] = jnp.zeros_like(cnt_ref)

    cs_ref[...] += csb
    cnt_ref[...] += cntb


def _segment_sums_tc(features, labels):
    k = _NT // _NB
    labels_b = labels[:_NT].reshape(k, 1, _NB)
    return pl.pallas_call(
        _seg_tc_body,
        grid=(k,),
        in_specs=[
            pl.BlockSpec((1, 1, _NB), lambda g: (g, 0, 0)),
            pl.BlockSpec((_NB, _F), lambda g: (g, 0)),
        ],
        out_specs=[
            pl.BlockSpec((_CP, _F), lambda g: (0, 0)),
            pl.BlockSpec((8, _CP), lambda g: (0, 0)),
        ],
        out_shape=[
            jax.ShapeDtypeStruct((_CP, _F), jnp.float32),
            jax.ShapeDtypeStruct((8, _CP), jnp.float32),
        ],
    )(labels_b, features)


def _epi_body(in_ref, idx_ref, lab2_ref, cs_ref, cnt_ref,
              cst_ref, cntt_ref, out_ref):
    b = in_ref.shape[0]
    cs = (cs_ref[0] + cs_ref[1] + cst_ref[...])[:_C]
    cntrow = (jnp.sum(cnt_ref[...], axis=0)
              + cntt_ref[0:1, :])[:, :_C]

    u = in_ref[...]
    ss = jnp.sum(u * u, axis=1, keepdims=True)
    un = u * lax.rsqrt(jnp.maximum(ss, 1e-24))
    logits = lax.dot_general(un, cs, (((1,), (1,)), ((), ())),
                             preferred_element_type=jnp.float32)
    mask = cntrow > 0.0
    denom = jnp.where(mask, cntrow, 1.0)
    sim = logits / (_TEMP * denom)
    exps = jnp.exp(sim) * mask.astype(jnp.float32)
    sums = jnp.sum(exps, axis=1, keepdims=True) + 1e-6
    logp = jnp.log(exps / sums + 1e-6)
    idx = idx_ref[...]
    q = idx // _C
    r = idx - q * _C
    l2 = lab2_ref[...].astype(jnp.float32)
    qio = lax.broadcasted_iota(jnp.int32, (b, l2.shape[0]), 1)
    ohq = (qio == q).astype(jnp.float32)
    rowsel = lax.dot_general(ohq, l2, (((1,), (0,)), ((), ())),
                             preferred_element_type=jnp.float32)
    rio = lax.broadcasted_iota(jnp.int32, (b, _C), 1)
    ohr = (rio == r).astype(jnp.float32)
    tcol = jnp.sum(rowsel * ohr, axis=1, keepdims=True)
    cio = lax.broadcasted_iota(jnp.int32, (b, _C), 1)
    oht = (cio == tcol.astype(jnp.int32)).astype(jnp.float32)
    loss = -jnp.sum(logp * oht) / float(b)
    out_ref[...] = jnp.full((1, 1), loss, jnp.float32)


def kernel(inputs, indexes, features, labels):
    n, f = features.shape
    b = inputs.shape[0]

    cs_parts, cnt_parts = _segment_sums_sc(features, labels)
    cs_tc, cnt_tc = _segment_sums_tc(features, labels)
    cnt3 = cnt_parts.reshape(32, 1, _CP)

    args = (inputs, indexes.reshape(b, 1), labels.reshape(n // _C, _C),
            cs_parts, cnt3, cs_tc, cnt_tc)
    loss = pl.pallas_call(
        _epi_body,
        in_specs=[pl.BlockSpec(a.shape, functools.partial(
                      lambda r: (0,) * r, len(a.shape))) for a in args],
        out_specs=pl.BlockSpec((1, 1), lambda: (0, 0)),
        out_shape=jax.ShapeDtypeStruct((1, 1), jnp.float32),
    )(*args)

    return loss[0, 0]


# speedup vs baseline: 1.0794x; 1.0007x over previous
import functools

import jax
import jax.numpy as jnp
from jax import lax
from jax.experimental import pallas as pl
from jax.experimental.pallas import tpu as pltpu
from jax.experimental.pallas import tpu_sc as plsc

_TEMP = 0.05
_C = 1000
_CP = 1024
_SL = 64
_N = 100000
_F = 128
_CH = 160
_NB = 4000
_NT = 56000
_NCHUNKS = (_N - _NT) // _CH
_SC_T0 = _NT // _CH
_NW = 32
_NG = 16
_BASE = _NCHUNKS // _NW
_REM = _NCHUNKS % _NW
_B = 1024


def _sc_body(feat_hbm, lab_hbm, cs_out, cnt_out,
             in0, in1, lb0, lb1, st0, st1, lbs0, lbs1, cnt,
             acc_sh,
             si0, si1, sl0, sl1, ss0, ss1):
    c = lax.axis_index("c")
    s = lax.axis_index("s")
    wid = s * 2 + c
    start = _SC_T0 + wid * _BASE + jnp.minimum(wid, _REM)
    nch = _BASE + jnp.where(wid < _REM, 1, 0)

    iota = lax.iota(jnp.int32, 16)
    zero16 = jnp.zeros((16,), jnp.float32)
    ones16 = jnp.ones((16,), jnp.float32)

    lane0 = iota == 0

    def zrow(i, _):
        for k in range(4):
            for j in range(_F // 16):
                st0[i * 4 + k, pl.ds(16 * j, 16)] = zero16
        return 0

    lax.fori_loop(0, _CH // 4, zrow, 0)
    pltpu.sync_copy(st0.at[pl.ds(0, _SL)], acc_sh.at[pl.ds(s * _SL, _SL)])
    for i in range(_CP // 16):
        cnt[pl.ds(16 * i, 16)] = zero16
    plsc.subcore_barrier()

    def _start(t, fbuf, lbuf, fsem, lsem):
        r0 = t * _CH
        pltpu.async_copy(feat_hbm.at[pl.ds(r0, _CH)], fbuf, fsem)
        pltpu.async_copy(lab_hbm.at[pl.ds(r0, _CH)], lbuf, lsem)

    def _wait_in(fbuf, lbuf, fsem, lsem):
        pltpu.make_async_copy(feat_hbm.at[pl.ds(0, _CH)], fbuf, fsem).wait()
        pltpu.make_async_copy(lab_hbm.at[pl.ds(0, _CH)], lbuf, lsem).wait()

    def _wait_scat(sbuf, lbsbuf, ssem):
        pltpu.make_async_copy(sbuf, acc_sh.at[lbsbuf], ssem).wait()

    def _fire_scat(sbuf, lbsbuf, ssem):
        pltpu.async_copy(sbuf, acc_sh.at[lbsbuf], ssem, add=True)

    def _process(fbuf, lbuf, sbuf, lbsbuf):
        for j in range(_CH // 16):
            lbsbuf[pl.ds(16 * j, 16)] = lbuf[pl.ds(16 * j, 16)]

        def sub(kb, _):
            labs16 = lbuf[pl.ds(kb * 16, 16)]
            for rr in range(16):
                r = kb * 16 + rr
                labsp = labs16.at[jnp.full((16,), rr, jnp.int32)].get(
                    mode="promise_in_bounds")
                plsc.addupdate_scatter(cnt, [labsp], ones16, mask=lane0)
                v = [fbuf[r, pl.ds(16 * j, 16)] for j in range(8)]
                sq = [x * x for x in v]
                tot = (((sq[0] + sq[1]) + (sq[2] + sq[3]))
                       + ((sq[4] + sq[5]) + (sq[6] + sq[7])))
                for k in (1, 2, 4, 8):
                    tot = tot + tot.at[iota ^ k].get(mode="promise_in_bounds")
                tot = jnp.maximum(tot, 1e-24)
                ib = lax.bitcast_convert_type(tot, jnp.int32)
                yi = jnp.int32(0x5F3759DF) - lax.shift_right_logical(ib, 1)
                yv = lax.bitcast_convert_type(yi, jnp.float32)
                h = 0.5 * tot
                for _ in range(1):
                    yv = yv * (1.5 - h * yv * yv)
                for j in range(8):
                    sbuf[r, pl.ds(16 * j, 16)] = v[j] * yv
            return 0

        lax.fori_loop(0, _CH // 16, sub, 0)

    _start(start, in0, lb0, si0, sl0)
    _start(start + 1, in1, lb1, si1, sl1)

    def pair(p, _):
        t0 = start + 2 * p
        _wait_in(in0, lb0, si0, sl0)

        @pl.when(p > 0)
        def _():
            _wait_scat(st0, lbs0, ss0)

        _process(in0, lb0, st0, lbs0)

        @pl.when(2 * p + 2 < nch)
        def _():
            _start(t0 + 2, in0, lb0, si0, sl0)

        _fire_scat(st0, lbs0, ss0)

        @pl.when(2 * p + 1 < nch)
        def _():
            _wait_in(in1, lb1, si1, sl1)

            @pl.when(p > 0)
            def _():
                _wait_scat(st1, lbs1, ss1)

            _process(in1, lb1, st1, lbs1)

            @pl.when(2 * p + 3 < nch)
            def _():
                _start(t0 + 3, in1, lb1, si1, sl1)

            _fire_scat(st1, lbs1, ss1)

        return 0

    lax.fori_loop(0, (nch + 1) // 2, pair, 0)

    _wait_scat(st0, lbs0, ss0)
    _wait_scat(st1, lbs1, ss1)
    plsc.subcore_barrier()

    pltpu.sync_copy(acc_sh.at[pl.ds(s * _SL, _SL)],
                    cs_out.at[c, pl.ds(s * _SL, _SL)])
    pltpu.sync_copy(cnt, cnt_out.at[c, s])


def _segment_sums_sc(features, labels):
    mesh = plsc.VectorSubcoreMesh(core_axis_name="c", subcore_axis_name="s")
    k = pl.kernel(
        _sc_body,
        out_type=[
            jax.ShapeDtypeStruct((2, _CP, _F), jnp.float32),
            jax.ShapeDtypeStruct((2, _NG, _CP), jnp.float32),
        ],
        mesh=mesh,
        scratch_types=[
            pltpu.VMEM((_CH, _F), jnp.float32),
            pltpu.VMEM((_CH, _F), jnp.float32),
            pltpu.VMEM((_CH,), jnp.int32),
            pltpu.VMEM((_CH,), jnp.int32),
            pltpu.VMEM((_CH, _F), jnp.float32),
            pltpu.VMEM((_CH, _F), jnp.float32),
            pltpu.VMEM((_CH,), jnp.int32),
            pltpu.VMEM((_CH,), jnp.int32),
            pltpu.VMEM((_CP,), jnp.float32),
            pltpu.VMEM_SHARED((_CP, _F), jnp.float32),
            pltpu.SemaphoreType.DMA,
            pltpu.SemaphoreType.DMA,
            pltpu.SemaphoreType.DMA,
            pltpu.SemaphoreType.DMA,
            pltpu.SemaphoreType.DMA,
            pltpu.SemaphoreType.DMA,
        ],
        compiler_params=pltpu.CompilerParams(needs_layout_passes=False),
    )
    return k(features, labels)


def _seg_tc_body(lab_ref, feat_ref, cs_ref, cnt_ref):
    g = pl.program_id(0)
    fb = feat_ref[...]
    ss = jnp.sum(fb * fb, axis=1, keepdims=True)
    rn = lax.rsqrt(jnp.maximum(ss, 1e-24))
    fn = (fb * rn).astype(jnp.bfloat16)
    lab = lab_ref[0]
    cio = lax.broadcasted_iota(jnp.int32, (_CP, _NB), 0)
    oh = (cio == lab).astype(jnp.bfloat16)
    csb = lax.dot_general(oh, fn, (((1,), (0,)), ((), ())),
                          preferred_element_type=jnp.float32)
    ones8 = jnp.ones((8, _NB), jnp.bfloat16)
    cntb = lax.dot_general(ones8, oh, (((1,), (1,)), ((), ())),
                           preferred_element_type=jnp.float32)

    @pl.when(g == 0)
    def _init():
        cs_ref[...] = jnp.zeros_like(cs_ref)
        cnt_ref[...] = jnp.zeros_like(cnt_ref)

    cs_ref[...] += csb
    cnt_ref[...] += cntb


def _segment_sums_tc(features, labels):
    k = _NT // _NB
    labels_b = labels[:_NT].reshape(k, 1, _NB)
    return pl.pallas_call(
        _seg_tc_body,
        grid=(k,),
        in_specs=[
            pl.BlockSpec((1, 1, _NB), lambda g: (g, 0, 0)),
            pl.BlockSpec((_NB, _F), lambda g: (g, 0)),
        ],
        out_specs=[
            pl.BlockSpec((_CP, _F), lambda g: (0, 0)),
            pl.BlockSpec((8, _CP), lambda g: (0, 0)),
        ],
        out_shape=[
            jax.ShapeDtypeStruct((_CP, _F), jnp.float32),
            jax.ShapeDtypeStruct((8, _CP), jnp.float32),
        ],
    )(labels_b, features)


def _epi_body(in_ref, idx_ref, lab2_ref, cs_ref, cnt_ref,
              cst_ref, cntt_ref, out_ref):
    b = in_ref.shape[0]
    cs = (cs_ref[0] + cs_ref[1] + cst_ref[...])[:_C]
    cntrow = (jnp.sum(cnt_ref[...], axis=0)
              + cntt_ref[0:1, :])[:, :_C]

    u = in_ref[...]
    ss = jnp.sum(u * u, axis=1, keepdims=True)
    un = u * lax.rsqrt(jnp.maximum(ss, 1e-24))
    logits = lax.dot_general(un, cs, (((1,), (1,)), ((), ())),
                             preferred_element_type=jnp.float32)
    mask = cntrow > 0.0
    denom = jnp.where(mask, cntrow, 1.0)
    sim = logits / (_TEMP * denom)
    exps = jnp.exp(sim) * mask.astype(jnp.float32)
    sums = jnp.sum(exps, axis=1, keepdims=True) + 1e-6
    logp = jnp.log(exps / sums + 1e-6)
    idx = idx_ref[...]
    q = idx // _C
    r = idx - q * _C
    l2 = lab2_ref[...].astype(jnp.float32)
    qio = lax.broadcasted_iota(jnp.int32, (b, l2.shape[0]), 1)
    ohq = (qio == q).astype(jnp.float32)
    rowsel = lax.dot_general(ohq, l2, (((1,), (0,)), ((), ())),
                             preferred_element_type=jnp.float32)
    rio = lax.broadcasted_iota(jnp.int32, (b, _C), 1)
    ohr = (rio == r).astype(jnp.float32)
    tcol = jnp.sum(rowsel * ohr, axis=1, keepdims=True)
    cio = lax.broadcasted_iota(jnp.int32, (b, _C), 1)
    oht = (cio == tcol.astype(jnp.int32)).astype(jnp.float32)
    loss = -jnp.sum(logp * oht) / float(b)
    out_ref[...] = jnp.full((1, 1), loss, jnp.float32)


def kernel(inputs, indexes, features, labels):
    n, f = features.shape
    b = inputs.shape[0]

    cs_parts, cnt_parts = _segment_sums_sc(features, labels)
    cs_tc, cnt_tc = _segment_sums_tc(features, labels)
    cnt3 = cnt_parts.reshape(32, 1, _CP)

    args = (inputs, indexes.reshape(b, 1), labels.reshape(n // _C, _C),
            cs_parts, cnt3, cs_tc, cnt_tc)
    loss = pl.pallas_call(
        _epi_body,
        in_specs=[pl.BlockSpec(a.shape, functools.partial(
                      lambda r: (0,) * r, len(a.shape))) for a in args],
        out_specs=pl.BlockSpec((1, 1), lambda: (0, 0)),
        out_shape=jax.ShapeDtypeStruct((1, 1), jnp.float32),
    )(*args)

    return loss[0, 0]
